# Initial kernel scaffold; baseline (speedup 1.0000x reference)
import functools

import jax
import jax.numpy as jnp
from jax import lax
from jax.experimental import pallas as pl
from jax.experimental.pallas import tpu as pltpu
from jax.experimental.pallas import tpu_sc as plsc

BATCH = 16384
HIST = 20
DIM = 32
TOTAL = BATCH * HIST

NC = 2
NS = 16
NW = NC * NS
BPW = TOTAL // NW

CH = 128
K = 8
GROUP = CH * K
GROUPS = BPW // GROUP

_mesh = plsc.VectorSubcoreMesh(core_axis_name="c", subcore_axis_name="s")


@functools.partial(
    pl.kernel,
    mesh=_mesh,
    out_type=jax.ShapeDtypeStruct((TOTAL, DIM), jnp.float32),
    scratch_types=[
        pltpu.VMEM((BPW // CH, CH), jnp.int32),
        pltpu.VMEM((2, GROUP, DIM), jnp.float32),
        pltpu.SemaphoreType.DMA,
        pltpu.SemaphoreType.DMA,
    ],
)
def _gather(idx_hbm, table_hbm, out_hbm, idx_v, rows_v, sem0, sem1):
    wid = lax.axis_index("s") * NC + lax.axis_index("c")
    pltpu.sync_copy(idx_hbm.at[wid], idx_v)

    sems = (sem0, sem1)
    handles = [None, None]

    def fire(g, buf):
        hs = []
        for b in range(K):
            hs.append(
                pltpu.async_copy(
                    table_hbm.at[idx_v.at[g * K + b]],
                    rows_v.at[buf].at[pl.ds(b * CH, CH)],
                    sems[buf],
                )
            )
        handles[buf] = hs

    fire(0, 0)
    for g in range(GROUPS):
        cur = g % 2
        if g + 1 < GROUPS:
            fire(g + 1, 1 - cur)
        for h in handles[cur]:
            h.wait()
        row0 = wid * BPW + g * GROUP
        pltpu.sync_copy(rows_v.at[cur], out_hbm.at[pl.ds(row0, GROUP)])


def kernel(xs, table):
    idx = xs.astype(jnp.int32).reshape(NW, BPW // CH, CH)
    out = _gather(idx, table)
    return out.reshape(BATCH, HIST, DIM)


# speedup vs baseline: 1.5117x; 1.5117x over previous
import functools

import jax
import jax.numpy as jnp
from jax import lax
from jax.experimental import pallas as pl
from jax.experimental.pallas import tpu as pltpu
from jax.experimental.pallas import tpu_sc as plsc

BATCH = 16384
HIST = 20
DIM = 32
TOTAL = BATCH * HIST

NC = 2
NS = 16
NW = NC * NS
BPW = TOTAL // NW

CH = 128
K = 8
GROUP = CH * K
GROUPS = BPW // GROUP

_mesh = plsc.VectorSubcoreMesh(core_axis_name="c", subcore_axis_name="s")


@functools.partial(
    pl.kernel,
    mesh=_mesh,
    out_type=jax.ShapeDtypeStruct((TOTAL, DIM), jnp.float32),
    scratch_types=[
        pltpu.VMEM((BPW // CH, CH), jnp.int32),
        pltpu.VMEM((2, GROUP, DIM), jnp.float32),
        pltpu.SemaphoreType.DMA,
        pltpu.SemaphoreType.DMA,
    ],
    compiler_params=pltpu.CompilerParams(use_tc_tiling_on_sc=False),
)
def _gather(idx_hbm, table_hbm, out_hbm, idx_v, rows_v, sem0, sem1):
    wid = lax.axis_index("s") * NC + lax.axis_index("c")
    pltpu.sync_copy(idx_hbm.at[wid], idx_v)

    sems = (sem0, sem1)
    handles = [None, None]

    def fire(g, buf):
        hs = []
        for b in range(K):
            hs.append(
                pltpu.async_copy(
                    table_hbm.at[idx_v.at[g * K + b]],
                    rows_v.at[buf].at[pl.ds(b * CH, CH)],
                    sems[buf],
                )
            )
        handles[buf] = hs

    fire(0, 0)
    for g in range(GROUPS):
        cur = g % 2
        if g + 1 < GROUPS:
            fire(g + 1, 1 - cur)
        for h in handles[cur]:
            h.wait()
        row0 = wid * BPW + g * GROUP
        pltpu.sync_copy(rows_v.at[cur], out_hbm.at[pl.ds(row0, GROUP)])


def kernel(xs, table):
    idx = xs.astype(jnp.int32).reshape(NW, BPW // CH, CH)
    out = _gather(idx, table)
    return out.reshape(BATCH, HIST, DIM)
